# Initial kernel scaffold; baseline (speedup 1.0000x reference)
import jax
import jax.numpy as jnp
from jax.experimental import pallas as pl


def kernel(data, iword_indicator, iword_numerals, ivectors_weight):
    raise NotImplementedError("write your pallas kernel here")


# speedup vs baseline: 1.8539x; 1.8539x over previous
import functools

import jax
import jax.numpy as jnp
from jax import lax
from jax.experimental import pallas as pl
from jax.experimental.pallas import tpu as pltpu
from jax.experimental.pallas import tpu_sc as plsc

VOCAB = 1000000
EMBED = 64
BATCH = 16384
HIST = 50

B = BATCH * HIST
NC, NS = 2, 16
NW = NC * NS
B_PER_W = B // NW
CHUNK = 512
NCHUNK = B_PER_W // CHUNK
NBUF = 2


def _make_gather():
    mesh = plsc.VectorSubcoreMesh(core_axis_name="c", subcore_axis_name="s")

    @functools.partial(
        pl.kernel,
        mesh=mesh,
        out_type=jax.ShapeDtypeStruct((B, EMBED), jnp.float32),
        compiler_params=pltpu.CompilerParams(use_tc_tiling_on_sc=False),
        scratch_types=(
            [pltpu.VMEM((CHUNK,), jnp.int32) for _ in range(NBUF)]
            + [pltpu.VMEM((CHUNK, EMBED), jnp.float32) for _ in range(NBUF)]
            + [pltpu.SemaphoreType.DMA for _ in range(NBUF)]
        ),
    )
    def gather_kernel(idx_hbm, table_hbm, out_hbm,
                      idx0, idx1, rows0, rows1, sem0, sem1):
        wid = lax.axis_index("s") * NC + lax.axis_index("c")
        base = wid * B_PER_W
        idx_v = [idx0, idx1]
        rows_v = [rows0, rows1]
        sems = [sem0, sem1]

        def start(chunk):
            slot = lax.rem(chunk, NBUF)
            off = base + chunk * CHUNK
            for s in range(NBUF):
                @pl.when(slot == s)
                def _():
                    pltpu.sync_copy(idx_hbm.at[pl.ds(off, CHUNK)], idx_v[s])
                    pltpu.async_copy(table_hbm.at[idx_v[s]], rows_v[s], sems[s])

        def drain(chunk):
            slot = lax.rem(chunk, NBUF)
            off = base + chunk * CHUNK
            for s in range(NBUF):
                @pl.when(slot == s)
                def _():
                    pltpu.make_async_copy(table_hbm.at[idx_v[s]], rows_v[s],
                                          sems[s]).wait()
                    pltpu.sync_copy(rows_v[s], out_hbm.at[pl.ds(off, CHUNK)])

        for p in range(NBUF - 1):
            start(p)

        def body(i, carry):
            start(i + NBUF - 1)
            drain(i)
            return carry

        lax.fori_loop(0, NCHUNK - (NBUF - 1), body, 0)
        for t in range(NCHUNK - (NBUF - 1), NCHUNK):
            drain(t)

    return gather_kernel


_gather = _make_gather()


def kernel(data, iword_indicator, iword_numerals, ivectors_weight):
    idx = data.reshape(-1).astype(jnp.int32)
    flat = _gather(idx, ivectors_weight)
    embed = flat.reshape(BATCH, HIST, EMBED)
    if iword_numerals.shape[0] == 0:
        return embed
    numerals = jnp.sign(iword_numerals) * jnp.log(jnp.abs(iword_numerals) + 1.0)
    ne = jnp.ones((EMBED, numerals.shape[0]), jnp.float32).at[0].set(numerals)
    ne = ne.T / (EMBED * 2)
    flat2 = embed.reshape(-1, EMBED)
    mask = iword_indicator.reshape(-1)
    pos = jnp.nonzero(mask, size=iword_numerals.shape[0])[0]
    return flat2.at[pos].set(ne).reshape(embed.shape)


# speedup vs baseline: 1.9492x; 1.0514x over previous
import functools

import jax
import jax.numpy as jnp
from jax import lax
from jax.experimental import pallas as pl
from jax.experimental.pallas import tpu as pltpu
from jax.experimental.pallas import tpu_sc as plsc

VOCAB = 1000000
EMBED = 64
BATCH = 16384
HIST = 50

B = BATCH * HIST
NC, NS = 2, 16
NW = NC * NS
CHUNK = 512
CPH = BATCH // CHUNK
NCHUNK = B // CHUNK
PER_W = NCHUNK // NW
NBUF = 2


def _make_gather():
    mesh = plsc.VectorSubcoreMesh(core_axis_name="c", subcore_axis_name="s")

    @functools.partial(
        pl.kernel,
        mesh=mesh,
        out_type=jax.ShapeDtypeStruct((HIST, BATCH, EMBED), jnp.float32),
        compiler_params=pltpu.CompilerParams(use_tc_tiling_on_sc=False),
        scratch_types=(
            [pltpu.VMEM((CHUNK,), jnp.int32) for _ in range(NBUF)]
            + [pltpu.VMEM((CHUNK, EMBED), jnp.float32) for _ in range(NBUF)]
            + [pltpu.SemaphoreType.DMA for _ in range(NBUF)]
        ),
    )
    def gather_kernel(idx_hbm, table_hbm, out_hbm,
                      idx0, idx1, rows0, rows1, gsem0, gsem1):
        wid = lax.axis_index("s") * NC + lax.axis_index("c")
        idx_v = [idx0, idx1]
        rows_v = [rows0, rows1]
        gsems = [gsem0, gsem1]

        def start(i, s):
            c = wid + i * NW
            off = c * CHUNK
            pltpu.sync_copy(idx_hbm.at[pl.ds(off, CHUNK)], idx_v[s])
            pltpu.async_copy(table_hbm.at[idx_v[s]], rows_v[s], gsems[s])

        def drain(i, s):
            c = wid + i * NW
            h = c // CPH
            b0 = (c % CPH) * CHUNK
            pltpu.make_async_copy(table_hbm.at[idx_v[s]], rows_v[s],
                                  gsems[s]).wait()
            pltpu.sync_copy(rows_v[s], out_hbm.at[h, pl.ds(b0, CHUNK)])

        start(0, 0)

        def body(i, carry):
            for s in range(NBUF):
                @pl.when((i % NBUF) == s)
                def _():
                    start(i, s)
            for s in range(NBUF):
                @pl.when(((i - 1) % NBUF) == s)
                def _():
                    drain(i - 1, s)
            return carry

        lax.fori_loop(1, PER_W, body, 0)
        for s in range(NBUF):
            @pl.when(((PER_W - 1) % NBUF) == s)
            def _():
                drain(PER_W - 1, s)

    return gather_kernel


_gather = _make_gather()


def kernel(data, iword_indicator, iword_numerals, ivectors_weight):
    idx = data.T.reshape(-1).astype(jnp.int32)
    out3 = _gather(idx, ivectors_weight)
    embed = out3.transpose(1, 0, 2)
    if iword_numerals.shape[0] == 0:
        return embed
    numerals = jnp.sign(iword_numerals) * jnp.log(jnp.abs(iword_numerals) + 1.0)
    ne = jnp.ones((EMBED, numerals.shape[0]), jnp.float32).at[0].set(numerals)
    ne = ne.T / (EMBED * 2)
    flat2 = embed.reshape(-1, EMBED)
    mask = iword_indicator.reshape(-1)
    pos = jnp.nonzero(mask, size=iword_numerals.shape[0])[0]
    return flat2.at[pos].set(ne).reshape(embed.shape)
